# baseline (speedup 1.0000x reference)
import functools

import jax
import jax.numpy as jnp
from jax import lax
from jax.experimental import pallas as pl
from jax.experimental.pallas import tpu as pltpu

_SLOPE = 0.1
_EPS = 1e-5


def _lrelu(v):
    return jnp.maximum(v, _SLOPE * v)


def _fill_pad(zp_ref, z, H, W, C):
    zp_ref[0, :, :] = jnp.zeros((W + 2, C), jnp.float32)
    zp_ref[H + 1, :, :] = jnp.zeros((W + 2, C), jnp.float32)
    zp_ref[1:H + 1, 0:1, :] = jnp.zeros((H, 1, C), jnp.float32)
    zp_ref[1:H + 1, W + 1:W + 2, :] = jnp.zeros((H, 1, C), jnp.float32)
    zp_ref[1:H + 1, 1:W + 1, :] = z.reshape(H, W, C)


def _taps(zp_ref, dw_ref, H, W):
    w = dw_ref[...]
    acc = zp_ref[0:H, 0:W, :] * w[0]
    for t in range(1, 9):
        i, j = divmod(t, 3)
        acc = acc + zp_ref[i:i + H, j:j + W, :] * w[t]
    return acc


def _finish(acc, pw_ref, y_ref, st_ref, HW, C):
    out = lax.dot_general(acc.reshape(HW, C), pw_ref[...],
                          (((1,), (0,)), ((), ())),
                          preferred_element_type=jnp.float32)
    st_ref[0] = jnp.stack([jnp.sum(out, axis=0), jnp.sum(out * out, axis=0)])
    y_ref[0] = out.astype(jnp.bfloat16)


def _s1_body(x_ref, dw_ref, pw_ref, y_ref, st_ref, zp_ref, *, H, W):
    C = x_ref.shape[1]
    HW = H * W
    row = lax.broadcasted_iota(jnp.int32, (C, C), 0)
    col = lax.broadcasted_iota(jnp.int32, (C, C), 1)
    eye = (row == col).astype(jnp.bfloat16)
    zt = lax.dot_general(x_ref[0], eye,
                         (((0,), (0,)), ((), ())),
                         preferred_element_type=jnp.float32)
    _fill_pad(zp_ref, zt, H, W, C)
    acc = _taps(zp_ref, dw_ref, H, W)
    _finish(acc, pw_ref, y_ref, st_ref, HW, C)


def _s2_body(zin_ref, sc_ref, sh_ref, dw_ref, pw_ref, y_ref, st_ref,
             zp_ref, *, H, W):
    C = zin_ref.shape[2]
    z = _lrelu(zin_ref[0].astype(jnp.float32) * sc_ref[0] + sh_ref[0])
    _fill_pad(zp_ref, z, H, W, C)
    acc = _taps(zp_ref, dw_ref, H, W)
    _finish(acc, pw_ref, y_ref, st_ref, H * W, C)


def _affine_params(st, count, g, b):
    tot = jnp.sum(st, axis=0)
    mean = tot[0] / count
    var = jnp.maximum(tot[1] / count - mean * mean, 0.0)
    scale = g.reshape(-1) * lax.rsqrt(var + _EPS)
    shift = b.reshape(-1) - mean * scale
    return scale, shift


def kernel(x_nchw, dw1, dw2, pw1, pw2, g1, b1, g2, b2):
    N, C, H, W = x_nchw.shape
    Cout = pw2.shape[1]
    HW = H * W
    d1 = dw1.reshape(9, C)
    d2 = dw2.reshape(9, C)

    zin = _lrelu(x_nchw).astype(jnp.bfloat16).reshape(N, C, HW)

    y1, st1 = pl.pallas_call(
        functools.partial(_s1_body, H=H, W=W),
        grid=(N,),
        in_specs=[
            pl.BlockSpec((1, C, HW), lambda b: (b, 0, 0)),
            pl.BlockSpec((9, C), lambda b: (0, 0)),
            pl.BlockSpec((C, C), lambda b: (0, 0)),
        ],
        out_specs=[
            pl.BlockSpec((1, HW, C), lambda b: (b, 0, 0)),
            pl.BlockSpec((1, 2, C), lambda b: (b, 0, 0)),
        ],
        out_shape=[
            jax.ShapeDtypeStruct((N, HW, C), jnp.bfloat16),
            jax.ShapeDtypeStruct((N, 2, C), jnp.float32),
        ],
        scratch_shapes=[pltpu.VMEM((H + 2, W + 2, C), jnp.float32)],
        compiler_params=pltpu.CompilerParams(
            dimension_semantics=("arbitrary",)),
    )(zin, d1, pw1)
    sc1, sh1 = _affine_params(st1, N * HW, g1, b1)

    y2, st2 = pl.pallas_call(
        functools.partial(_s2_body, H=H, W=W),
        grid=(N,),
        in_specs=[
            pl.BlockSpec((1, HW, C), lambda b: (b, 0, 0)),
            pl.BlockSpec((1, C), lambda b: (0, 0)),
            pl.BlockSpec((1, C), lambda b: (0, 0)),
            pl.BlockSpec((9, C), lambda b: (0, 0)),
            pl.BlockSpec((C, Cout), lambda b: (0, 0)),
        ],
        out_specs=[
            pl.BlockSpec((1, HW, Cout), lambda b: (b, 0, 0)),
            pl.BlockSpec((1, 2, Cout), lambda b: (b, 0, 0)),
        ],
        out_shape=[
            jax.ShapeDtypeStruct((N, HW, Cout), jnp.bfloat16),
            jax.ShapeDtypeStruct((N, 2, Cout), jnp.float32),
        ],
        scratch_shapes=[pltpu.VMEM((H + 2, W + 2, C), jnp.float32)],
        compiler_params=pltpu.CompilerParams(
            dimension_semantics=("arbitrary",)),
    )(y1, sc1.reshape(1, C), sh1.reshape(1, C), d2, pw2)
    sc2, sh2 = _affine_params(st2, N * HW, g2, b2)

    out = y2.reshape(N, H, W, Cout).astype(jnp.float32)
    out = out * sc2.reshape(1, 1, 1, Cout) + sh2.reshape(1, 1, 1, Cout)
    return jnp.transpose(out, (0, 3, 1, 2))


# speedup vs baseline: 1.1362x; 1.1362x over previous
import functools

import jax
import jax.numpy as jnp
from jax import lax
from jax.experimental import pallas as pl
from jax.experimental.pallas import tpu as pltpu

_SLOPE = 0.1
_EPS = 1e-5
_BB = 2


def _lrelu(v):
    return jnp.maximum(v, _SLOPE * v)


def _fill_pad(zp_ref, z, H, W, C):
    zp_ref[0, :, :] = jnp.zeros((W + 2, C), jnp.float32)
    zp_ref[H + 1, :, :] = jnp.zeros((W + 2, C), jnp.float32)
    zp_ref[1:H + 1, 0:1, :] = jnp.zeros((H, 1, C), jnp.float32)
    zp_ref[1:H + 1, W + 1:W + 2, :] = jnp.zeros((H, 1, C), jnp.float32)
    zp_ref[1:H + 1, 1:W + 1, :] = z.reshape(H, W, C)


def _taps(zp_ref, dw_ref, H, W):
    w = dw_ref[...]
    acc = zp_ref[0:H, 0:W, :] * w[0]
    for t in range(1, 9):
        i, j = divmod(t, 3)
        acc = acc + zp_ref[i:i + H, j:j + W, :] * w[t]
    return acc


def _stage_body(zin_ref, sc_ref, sh_ref, dw_ref, pw_ref, y_ref, st_ref,
                zp_ref, *, H, W, affine):
    C = zin_ref.shape[2]
    HW = H * W
    for img in range(_BB):
        z = zin_ref[img].astype(jnp.float32)
        if affine:
            z = _lrelu(z * sc_ref[0] + sh_ref[0])
        _fill_pad(zp_ref, z, H, W, C)
        acc = _taps(zp_ref, dw_ref, H, W)
        out = lax.dot_general(acc.reshape(HW, C), pw_ref[...],
                              (((1,), (0,)), ((), ())),
                              preferred_element_type=jnp.float32)
        st_ref[img] = jnp.stack([jnp.sum(out, axis=0),
                                 jnp.sum(out * out, axis=0)])
        y_ref[img] = out.astype(jnp.bfloat16)


def _stage(zin, sc, sh, dw, pw, H, W, affine):
    N, HW, C = zin.shape
    Cout = pw.shape[1]
    return pl.pallas_call(
        functools.partial(_stage_body, H=H, W=W, affine=affine),
        grid=(N // _BB,),
        in_specs=[
            pl.BlockSpec((_BB, HW, C), lambda b: (b, 0, 0)),
            pl.BlockSpec((1, C), lambda b: (0, 0)),
            pl.BlockSpec((1, C), lambda b: (0, 0)),
            pl.BlockSpec((9, C), lambda b: (0, 0)),
            pl.BlockSpec((C, Cout), lambda b: (0, 0)),
        ],
        out_specs=[
            pl.BlockSpec((_BB, HW, Cout), lambda b: (b, 0, 0)),
            pl.BlockSpec((_BB, 2, Cout), lambda b: (b, 0, 0)),
        ],
        out_shape=[
            jax.ShapeDtypeStruct((N, HW, Cout), jnp.bfloat16),
            jax.ShapeDtypeStruct((N, 2, Cout), jnp.float32),
        ],
        scratch_shapes=[pltpu.VMEM((H + 2, W + 2, C), jnp.float32)],
        compiler_params=pltpu.CompilerParams(
            dimension_semantics=("arbitrary",)),
    )(zin, sc, sh, dw, pw)


def _affine_params(st, count, g, b):
    tot = jnp.sum(st, axis=0)
    mean = tot[0] / count
    var = jnp.maximum(tot[1] / count - mean * mean, 0.0)
    scale = g.reshape(-1) * lax.rsqrt(var + _EPS)
    shift = b.reshape(-1) - mean * scale
    return scale, shift


def kernel(x_nchw, dw1, dw2, pw1, pw2, g1, b1, g2, b2):
    N, C, H, W = x_nchw.shape
    Cout = pw2.shape[1]
    HW = H * W
    d1 = dw1.reshape(9, C)
    d2 = dw2.reshape(9, C)
    ones = jnp.ones((1, C), jnp.float32)
    zeros = jnp.zeros((1, C), jnp.float32)

    zt = jnp.transpose(_lrelu(x_nchw).astype(jnp.bfloat16), (0, 2, 3, 1))
    zt = zt.reshape(N, HW, C)

    y1, st1 = _stage(zt, ones, zeros, d1, pw1, H, W, affine=False)
    sc1, sh1 = _affine_params(st1, N * HW, g1, b1)

    y2, st2 = _stage(y1, sc1.reshape(1, C), sh1.reshape(1, C), d2, pw2, H, W,
                     affine=True)
    sc2, sh2 = _affine_params(st2, N * HW, g2, b2)

    out = y2.reshape(N, H, W, Cout).astype(jnp.float32)
    out = out * sc2.reshape(1, 1, 1, Cout) + sh2.reshape(1, 1, 1, Cout)
    return jnp.transpose(out, (0, 3, 1, 2))


# speedup vs baseline: 1.1449x; 1.0076x over previous
import functools

import jax
import jax.numpy as jnp
from jax import lax
from jax.experimental import pallas as pl
from jax.experimental.pallas import tpu as pltpu

_SLOPE = 0.1
_EPS = 1e-5
_BB = 4


def _lrelu(v):
    return jnp.maximum(v, _SLOPE * v)


def _fill_pad(zp_ref, z, H, W, C):
    zp_ref[0, :, :] = jnp.zeros((W + 2, C), jnp.float32)
    zp_ref[H + 1, :, :] = jnp.zeros((W + 2, C), jnp.float32)
    zp_ref[1:H + 1, 0:1, :] = jnp.zeros((H, 1, C), jnp.float32)
    zp_ref[1:H + 1, W + 1:W + 2, :] = jnp.zeros((H, 1, C), jnp.float32)
    zp_ref[1:H + 1, 1:W + 1, :] = z.reshape(H, W, C)


def _taps(zp_ref, dw_ref, H, W):
    w = dw_ref[...]
    acc = zp_ref[0:H, 0:W, :] * w[0]
    for t in range(1, 9):
        i, j = divmod(t, 3)
        acc = acc + zp_ref[i:i + H, j:j + W, :] * w[t]
    return acc


def _stage_body(zin_ref, sc_ref, sh_ref, dw_ref, pw_ref, y_ref, st_ref,
                zp_ref, *, H, W, affine, bb):
    C = zin_ref.shape[2]
    HW = H * W
    for img in range(bb):
        z = zin_ref[img].astype(jnp.float32)
        if affine:
            z = _lrelu(z * sc_ref[0] + sh_ref[0])
        _fill_pad(zp_ref, z, H, W, C)
        acc = _taps(zp_ref, dw_ref, H, W)
        out = lax.dot_general(acc.reshape(HW, C), pw_ref[...],
                              (((1,), (0,)), ((), ())),
                              preferred_element_type=jnp.float32)
        st_ref[img, 0, :] = jnp.sum(out, axis=0)
        st_ref[img, 1, :] = jnp.sum(out * out, axis=0)
        y_ref[img] = out.astype(jnp.bfloat16)


def _stage(zin, sc, sh, dw, pw, H, W, affine):
    N, HW, C = zin.shape
    Cout = pw.shape[1]
    bb = next(b for b in (_BB, 2, 1) if N % b == 0)
    return pl.pallas_call(
        functools.partial(_stage_body, H=H, W=W, affine=affine, bb=bb),
        grid=(N // bb,),
        in_specs=[
            pl.BlockSpec((bb, HW, C), lambda b: (b, 0, 0)),
            pl.BlockSpec((1, C), lambda b: (0, 0)),
            pl.BlockSpec((1, C), lambda b: (0, 0)),
            pl.BlockSpec((9, C), lambda b: (0, 0)),
            pl.BlockSpec((C, Cout), lambda b: (0, 0)),
        ],
        out_specs=[
            pl.BlockSpec((bb, HW, Cout), lambda b: (b, 0, 0)),
            pl.BlockSpec((bb, 2, Cout), lambda b: (b, 0, 0)),
        ],
        out_shape=[
            jax.ShapeDtypeStruct((N, HW, Cout), jnp.bfloat16),
            jax.ShapeDtypeStruct((N, 2, Cout), jnp.float32),
        ],
        scratch_shapes=[pltpu.VMEM((H + 2, W + 2, C), jnp.float32)],
        compiler_params=pltpu.CompilerParams(
            dimension_semantics=("arbitrary",)),
    )(zin, sc, sh, dw, pw)


def _affine_params(st, count, g, b):
    tot = jnp.sum(st, axis=0)
    mean = tot[0] / count
    var = jnp.maximum(tot[1] / count - mean * mean, 0.0)
    scale = g.reshape(-1) * lax.rsqrt(var + _EPS)
    shift = b.reshape(-1) - mean * scale
    return scale, shift


def kernel(x_nchw, dw1, dw2, pw1, pw2, g1, b1, g2, b2):
    N, C, H, W = x_nchw.shape
    Cout = pw2.shape[1]
    HW = H * W
    d1 = dw1.reshape(9, C)
    d2 = dw2.reshape(9, C)
    ones = jnp.ones((1, C), jnp.float32)
    zeros = jnp.zeros((1, C), jnp.float32)

    zt = jnp.transpose(_lrelu(x_nchw).astype(jnp.bfloat16), (0, 2, 3, 1))
    zt = zt.reshape(N, HW, C)

    y1, st1 = _stage(zt, ones, zeros, d1, pw1, H, W, affine=False)
    sc1, sh1 = _affine_params(st1, N * HW, g1, b1)

    y2, st2 = _stage(y1, sc1.reshape(1, C), sh1.reshape(1, C), d2, pw2, H, W,
                     affine=True)
    sc2, sh2 = _affine_params(st2, N * HW, g2, b2)

    out = y2.reshape(N, H, W, Cout).astype(jnp.float32)
    out = out * sc2.reshape(1, 1, 1, Cout) + sh2.reshape(1, 1, 1, Cout)
    return jnp.transpose(out, (0, 3, 1, 2))
